# baseline (speedup 1.0000x reference)
import functools

import jax
import jax.numpy as jnp
from jax import lax
from jax.experimental import pallas as pl
from jax.experimental.pallas import tpu as pltpu
from jax.experimental.pallas import tpu_sc as plsc

_NUM_CORES = 2
_NUM_SUBCORES = 16
_NW = _NUM_CORES * _NUM_SUBCORES

_SB = 8
_BB = 128


def _gather_t(table_p, idx_t, n_s, d, n_b):
    assert n_b // _BB == _NW
    w = 2 * d
    mesh = plsc.VectorSubcoreMesh(core_axis_name="c", subcore_axis_name="s")

    @functools.partial(
        pl.kernel,
        mesh=mesh,
        out_type=jax.ShapeDtypeStruct((n_s, d, n_b), jnp.float32),
        compiler_params=pltpu.CompilerParams(needs_layout_passes=False),
        scratch_types=[
            pltpu.VMEM((_SB, _BB), jnp.int32),
            pltpu.VMEM((2, _BB, w), jnp.float32),
            pltpu.SemaphoreType.DMA((2,)),
            pltpu.SemaphoreType.DMA((2,)),
        ],
    )
    def g_kernel(table_hbm, idx_hbm, out_hbm, idxt_v, rows_v, sem_g, sem_wb):
        wid = lax.axis_index("s") * _NUM_CORES + lax.axis_index("c")
        b0 = wid * _BB

        def fire(t):
            slot = lax.rem(t, 2)
            si = lax.rem(t, _SB)

            @pl.when(si == 0)
            def _stage():
                ts = pl.multiple_of(t, _SB)
                pltpu.sync_copy(
                    idx_hbm.at[pl.ds(ts, _SB), pl.ds(b0, _BB)], idxt_v)

            pltpu.async_copy(table_hbm.at[idxt_v.at[si]], rows_v.at[slot],
                             sem_g.at[slot])

        fire(0)

        def body(t, carry):
            slot = lax.rem(t, 2)
            pltpu.make_async_copy(table_hbm.at[pl.ds(0, _BB)],
                                  rows_v.at[slot], sem_g.at[slot]).wait()
            for dcol in range(d):
                pltpu.async_copy(rows_v.at[slot, :, dcol],
                                 out_hbm.at[t, dcol, pl.ds(b0, _BB)],
                                 sem_wb.at[slot])

            @pl.when(t >= 1)
            def _drain_prev_wb():
                pltpu.make_async_copy(
                    out_hbm.at[0, :, pl.ds(b0, _BB)],
                    rows_v.at[1 - slot, pl.ds(0, d)],
                    sem_wb.at[1 - slot]).wait()

            @pl.when(t < n_s - 1)
            def _prefetch():
                fire(t + 1)

            return carry

        lax.fori_loop(0, n_s, body, 0)
        pltpu.make_async_copy(out_hbm.at[0, :, pl.ds(b0, _BB)],
                              rows_v.at[lax.rem(n_s - 1, 2), pl.ds(0, d)],
                              sem_wb.at[lax.rem(n_s - 1, 2)]).wait()

    return g_kernel(table_p, idx_t)


def kernel(position_labels, pos_embedding_weight):
    b, s = position_labels.shape
    v, d = pos_embedding_weight.shape
    idx_t = position_labels.T.astype(jnp.int32)
    table_p = jnp.pad(pos_embedding_weight, ((0, 0), (0, d)))
    out_t = _gather_t(table_p, idx_t, s, d, b)
    return out_t.transpose(2, 0, 1)


# speedup vs baseline: 163.8001x; 163.8001x over previous
import functools

import jax
import jax.numpy as jnp
from jax import lax
from jax.experimental import pallas as pl
from jax.experimental.pallas import tpu as pltpu
from jax.experimental.pallas import tpu_sc as plsc

_NUM_CORES = 2
_NUM_SUBCORES = 16
_NW = _NUM_CORES * _NUM_SUBCORES

_C = 128
_K = 5


def _gather_rows(table, idx_2d, n_rows, d):
    gc = _C * _K
    b_per_w = n_rows // _NW
    n_groups = b_per_w // gc
    chunks_per_w = b_per_w // _C
    mesh = plsc.VectorSubcoreMesh(core_axis_name="c", subcore_axis_name="s")

    @functools.partial(
        pl.kernel,
        mesh=mesh,
        out_type=jax.ShapeDtypeStruct((n_rows, d), jnp.float32),
        compiler_params=pltpu.CompilerParams(use_tc_tiling_on_sc=False),
        scratch_types=[
            pltpu.VMEM((2, _K, _C), jnp.int32),
            pltpu.VMEM((2, gc, d), jnp.float32),
            pltpu.SemaphoreType.DMA((2,)),
        ],
    )
    def gather_kernel(table_hbm, idx_hbm, out_hbm, idx_v, rows_v, sem):
        wid = lax.axis_index("s") * _NUM_CORES + lax.axis_index("c")
        row_base = wid * b_per_w
        chunk_base = wid * chunks_per_w

        def fire(g, buf):
            pltpu.sync_copy(idx_hbm.at[pl.ds(chunk_base + g * _K, _K)],
                            idx_v.at[buf])
            for j in range(_K):
                pltpu.async_copy(table_hbm.at[idx_v.at[buf, j]],
                                 rows_v.at[buf, pl.ds(j * _C, _C)],
                                 sem.at[buf])

        def drain(g, buf):
            pltpu.make_async_copy(table_hbm.at[pl.ds(0, gc)],
                                  rows_v.at[buf], sem.at[buf]).wait()
            pltpu.sync_copy(rows_v.at[buf],
                            out_hbm.at[pl.ds(row_base + g * gc, gc)])

        fire(0, 0)

        def body(g, carry):
            buf = lax.rem(g, 2)
            fire(g + 1, 1 - buf)
            drain(g, buf)
            return carry

        lax.fori_loop(0, n_groups - 1, body, 0)
        drain(n_groups - 1, (n_groups - 1) % 2)

    return gather_kernel(table, idx_2d)


def kernel(position_labels, pos_embedding_weight):
    b, s = position_labels.shape
    v, d = pos_embedding_weight.shape
    n_rows = b * s
    idx_2d = position_labels.reshape(n_rows // _C, _C).astype(jnp.int32)
    out = _gather_rows(pos_embedding_weight, idx_2d, n_rows, d)
    return out.reshape(b, s, d)
